# Initial kernel scaffold; baseline (speedup 1.0000x reference)
import jax
import jax.numpy as jnp
from jax.experimental import pallas as pl


def kernel(x, Z, scale, shift):
    raise NotImplementedError("write your pallas kernel here")


# speedup vs baseline: 36.8325x; 36.8325x over previous
import functools

import jax
import jax.numpy as jnp
from jax import lax
from jax.experimental import pallas as pl
from jax.experimental.pallas import tpu as pltpu
from jax.experimental.pallas import tpu_sc as plsc

LANES = 16
NW = 32
B_PER_W = 3136
N_PAD = NW * B_PER_W
NV = B_PER_W // LANES
TAB = 128

_mesh = plsc.VectorSubcoreMesh(core_axis_name="c", subcore_axis_name="s")


@functools.partial(
    pl.kernel,
    mesh=_mesh,
    out_type=jax.ShapeDtypeStruct((N_PAD,), jnp.float32),
    scratch_types=[
        pltpu.VMEM((B_PER_W,), jnp.float32),
        pltpu.VMEM((B_PER_W,), jnp.int32),
        pltpu.VMEM((B_PER_W,), jnp.float32),
        pltpu.VMEM((TAB,), jnp.float32),
        pltpu.VMEM((TAB,), jnp.float32),
    ],
    compiler_params=pltpu.CompilerParams(needs_layout_passes=False),
)
def _scale_shift_sc(x_hbm, z_hbm, scale_hbm, shift_hbm, out_hbm,
                    x_v, z_v, o_v, sc_v, sh_v):
    wid = lax.axis_index("s") * 2 + lax.axis_index("c")
    base = wid * B_PER_W
    pltpu.sync_copy(scale_hbm, sc_v)
    pltpu.sync_copy(shift_hbm, sh_v)
    pltpu.sync_copy(x_hbm.at[pl.ds(base, B_PER_W)], x_v)
    pltpu.sync_copy(z_hbm.at[pl.ds(base, B_PER_W)], z_v)

    def body(i, carry):
        off = i * LANES
        z = z_v[pl.ds(off, LANES)]
        xv = x_v[pl.ds(off, LANES)]
        s = plsc.load_gather(sc_v, [z])
        t = plsc.load_gather(sh_v, [z])
        o_v[pl.ds(off, LANES)] = s * xv + t
        return carry

    lax.fori_loop(0, NV, body, 0)
    pltpu.sync_copy(o_v, out_hbm.at[pl.ds(base, B_PER_W)])


def kernel(x, Z, scale, shift):
    n = x.shape[0]
    xf = jnp.pad(x.reshape(-1), (0, N_PAD - n))
    zf = jnp.pad(Z.astype(jnp.int32), (0, N_PAD - n))
    sc = jnp.pad(scale.reshape(-1), (0, TAB - scale.shape[0]))
    sh = jnp.pad(shift.reshape(-1), (0, TAB - shift.shape[0]))
    out = _scale_shift_sc(xf, zf, sc, sh)
    return out[:n].reshape(x.shape)


# speedup vs baseline: 40.6861x; 1.1046x over previous
import functools

import jax
import jax.numpy as jnp
from jax import lax
from jax.experimental import pallas as pl
from jax.experimental.pallas import tpu as pltpu
from jax.experimental.pallas import tpu_sc as plsc

LANES = 16
NW = 32
N = 100000
B_PER_W = 3136
B_LAST = N - (NW - 1) * B_PER_W
NV = B_PER_W // LANES
NV_LAST = B_LAST // LANES
N_SPECIES = 119

_mesh = plsc.VectorSubcoreMesh(core_axis_name="c", subcore_axis_name="s")


@functools.partial(
    pl.kernel,
    mesh=_mesh,
    out_type=jax.ShapeDtypeStruct((N,), jnp.float32),
    scratch_types=[
        pltpu.VMEM((B_PER_W,), jnp.float32),
        pltpu.VMEM((B_PER_W,), jnp.int32),
        pltpu.VMEM((B_PER_W,), jnp.float32),
        pltpu.VMEM((N_SPECIES,), jnp.float32),
        pltpu.VMEM((N_SPECIES,), jnp.float32),
    ],
    compiler_params=pltpu.CompilerParams(needs_layout_passes=False),
)
def _scale_shift_sc(x_hbm, z_hbm, scale_hbm, shift_hbm, out_hbm,
                    x_v, z_v, o_v, sc_v, sh_v):
    wid = lax.axis_index("s") * 2 + lax.axis_index("c")
    base = wid * B_PER_W
    last = wid == NW - 1
    pltpu.sync_copy(scale_hbm, sc_v)
    pltpu.sync_copy(shift_hbm, sh_v)

    @pl.when(jnp.logical_not(last))
    def _():
        pltpu.sync_copy(x_hbm.at[pl.ds(base, B_PER_W)], x_v)
        pltpu.sync_copy(z_hbm.at[pl.ds(base, B_PER_W)], z_v)

    @pl.when(last)
    def _():
        pltpu.sync_copy(x_hbm.at[pl.ds(N - B_LAST, B_LAST)], x_v.at[pl.ds(0, B_LAST)])
        pltpu.sync_copy(z_hbm.at[pl.ds(N - B_LAST, B_LAST)], z_v.at[pl.ds(0, B_LAST)])

    def body(i, carry):
        off = i * LANES
        z = z_v[pl.ds(off, LANES)]
        xv = x_v[pl.ds(off, LANES)]
        s = plsc.load_gather(sc_v, [z])
        t = plsc.load_gather(sh_v, [z])
        o_v[pl.ds(off, LANES)] = s * xv + t
        return carry

    nv = jnp.where(last, NV_LAST, NV)
    lax.fori_loop(0, nv, body, 0)

    @pl.when(jnp.logical_not(last))
    def _():
        pltpu.sync_copy(o_v, out_hbm.at[pl.ds(base, B_PER_W)])

    @pl.when(last)
    def _():
        pltpu.sync_copy(o_v.at[pl.ds(0, B_LAST)], out_hbm.at[pl.ds(N - B_LAST, B_LAST)])


def kernel(x, Z, scale, shift):
    out = _scale_shift_sc(x.reshape(-1), Z.astype(jnp.int32),
                          scale.reshape(-1), shift.reshape(-1))
    return out.reshape(x.shape)


# speedup vs baseline: 46.1674x; 1.1347x over previous
import functools

import jax
import jax.numpy as jnp
from jax import lax
from jax.experimental import pallas as pl
from jax.experimental.pallas import tpu as pltpu
from jax.experimental.pallas import tpu_sc as plsc

LANES = 16
NW = 32
N = 100000
B_PER_W = 3136
B_LAST = N - (NW - 1) * B_PER_W
NV = B_PER_W // LANES
NV_LAST = B_LAST // LANES
N_SPECIES = 119

_mesh = plsc.VectorSubcoreMesh(core_axis_name="c", subcore_axis_name="s")


@functools.partial(
    pl.kernel,
    mesh=_mesh,
    out_type=jax.ShapeDtypeStruct((N,), jnp.float32),
    scratch_types=[
        pltpu.VMEM((B_PER_W,), jnp.float32),
        pltpu.VMEM((B_PER_W,), jnp.int32),
        pltpu.VMEM((B_PER_W,), jnp.float32),
        pltpu.VMEM((N_SPECIES,), jnp.float32),
        pltpu.VMEM((N_SPECIES,), jnp.float32),
        pltpu.SemaphoreType.DMA,
    ],
    compiler_params=pltpu.CompilerParams(needs_layout_passes=False),
)
def _scale_shift_sc(x_hbm, z_hbm, scale_hbm, shift_hbm, out_hbm,
                    x_v, z_v, o_v, sc_v, sh_v, sem):
    wid = lax.axis_index("s") * 2 + lax.axis_index("c")
    base = wid * B_PER_W
    last = wid == NW - 1
    c_sc = pltpu.async_copy(scale_hbm, sc_v, sem)
    c_sh = pltpu.async_copy(shift_hbm, sh_v, sem)

    @pl.when(jnp.logical_not(last))
    def _():
        pltpu.async_copy(x_hbm.at[pl.ds(base, B_PER_W)], x_v, sem)
        pltpu.async_copy(z_hbm.at[pl.ds(base, B_PER_W)], z_v, sem)
        pltpu.make_async_copy(x_hbm.at[pl.ds(base, B_PER_W)], x_v, sem).wait()
        pltpu.make_async_copy(z_hbm.at[pl.ds(base, B_PER_W)], z_v, sem).wait()

    @pl.when(last)
    def _():
        pltpu.async_copy(x_hbm.at[pl.ds(N - B_LAST, B_LAST)], x_v.at[pl.ds(0, B_LAST)], sem)
        pltpu.async_copy(z_hbm.at[pl.ds(N - B_LAST, B_LAST)], z_v.at[pl.ds(0, B_LAST)], sem)
        pltpu.make_async_copy(x_hbm.at[pl.ds(N - B_LAST, B_LAST)], x_v.at[pl.ds(0, B_LAST)], sem).wait()
        pltpu.make_async_copy(z_hbm.at[pl.ds(N - B_LAST, B_LAST)], z_v.at[pl.ds(0, B_LAST)], sem).wait()

    c_sc.wait()
    c_sh.wait()

    nv = jnp.where(last, NV_LAST, NV)

    @plsc.parallel_loop(0, nv, unroll=4)
    def _(i):
        off = i * LANES
        z = z_v[pl.ds(off, LANES)]
        xv = x_v[pl.ds(off, LANES)]
        s = plsc.load_gather(sc_v, [z])
        t = plsc.load_gather(sh_v, [z])
        o_v[pl.ds(off, LANES)] = s * xv + t

    @pl.when(jnp.logical_not(last))
    def _():
        pltpu.sync_copy(o_v, out_hbm.at[pl.ds(base, B_PER_W)])

    @pl.when(last)
    def _():
        pltpu.sync_copy(o_v.at[pl.ds(0, B_LAST)], out_hbm.at[pl.ds(N - B_LAST, B_LAST)])


def kernel(x, Z, scale, shift):
    out = _scale_shift_sc(x.reshape(-1), Z.astype(jnp.int32),
                          scale.reshape(-1), shift.reshape(-1))
    return out.reshape(x.shape)


# speedup vs baseline: 46.3459x; 1.0039x over previous
import functools

import jax
import jax.numpy as jnp
from jax import lax
from jax.experimental import pallas as pl
from jax.experimental.pallas import tpu as pltpu
from jax.experimental.pallas import tpu_sc as plsc

LANES = 16
NW = 32
N = 100000
B_PER_W = 3136
B_LAST = N - (NW - 1) * B_PER_W
NV = B_PER_W // LANES
NV_LAST = B_LAST // LANES
N_SPECIES = 119

_mesh = plsc.VectorSubcoreMesh(core_axis_name="c", subcore_axis_name="s")


@functools.partial(
    pl.kernel,
    mesh=_mesh,
    out_type=jax.ShapeDtypeStruct((N,), jnp.float32),
    scratch_types=[
        pltpu.VMEM((B_PER_W,), jnp.float32),
        pltpu.VMEM((B_PER_W,), jnp.int32),
        pltpu.VMEM((B_PER_W,), jnp.float32),
        pltpu.VMEM((N_SPECIES,), jnp.float32),
        pltpu.VMEM((N_SPECIES,), jnp.float32),
        pltpu.SemaphoreType.DMA,
    ],
    compiler_params=pltpu.CompilerParams(needs_layout_passes=False),
)
def _scale_shift_sc(x_hbm, z_hbm, scale_hbm, shift_hbm, out_hbm,
                    x_v, z_v, o_v, sc_v, sh_v, sem):
    wid = lax.axis_index("s") * 2 + lax.axis_index("c")
    base = wid * B_PER_W
    last = wid == NW - 1
    c_sc = pltpu.async_copy(scale_hbm, sc_v, sem)
    c_sh = pltpu.async_copy(shift_hbm, sh_v, sem)

    @pl.when(jnp.logical_not(last))
    def _():
        pltpu.async_copy(x_hbm.at[pl.ds(base, B_PER_W)], x_v, sem)
        pltpu.async_copy(z_hbm.at[pl.ds(base, B_PER_W)], z_v, sem)
        pltpu.make_async_copy(x_hbm.at[pl.ds(base, B_PER_W)], x_v, sem).wait()
        pltpu.make_async_copy(z_hbm.at[pl.ds(base, B_PER_W)], z_v, sem).wait()

    @pl.when(last)
    def _():
        pltpu.async_copy(x_hbm.at[pl.ds(N - B_LAST, B_LAST)], x_v.at[pl.ds(0, B_LAST)], sem)
        pltpu.async_copy(z_hbm.at[pl.ds(N - B_LAST, B_LAST)], z_v.at[pl.ds(0, B_LAST)], sem)
        pltpu.make_async_copy(x_hbm.at[pl.ds(N - B_LAST, B_LAST)], x_v.at[pl.ds(0, B_LAST)], sem).wait()
        pltpu.make_async_copy(z_hbm.at[pl.ds(N - B_LAST, B_LAST)], z_v.at[pl.ds(0, B_LAST)], sem).wait()

    c_sc.wait()
    c_sh.wait()

    nv = jnp.where(last, NV_LAST, NV)

    @plsc.parallel_loop(0, nv, unroll=8)
    def _(i):
        off = i * LANES
        z = z_v[pl.ds(off, LANES)]
        xv = x_v[pl.ds(off, LANES)]
        s = plsc.load_gather(sc_v, [z])
        t = plsc.load_gather(sh_v, [z])
        o_v[pl.ds(off, LANES)] = s * xv + t

    @pl.when(jnp.logical_not(last))
    def _():
        pltpu.sync_copy(o_v, out_hbm.at[pl.ds(base, B_PER_W)])

    @pl.when(last)
    def _():
        pltpu.sync_copy(o_v.at[pl.ds(0, B_LAST)], out_hbm.at[pl.ds(N - B_LAST, B_LAST)])


def kernel(x, Z, scale, shift):
    out = _scale_shift_sc(x.reshape(-1), Z.astype(jnp.int32),
                          scale.reshape(-1), shift.reshape(-1))
    return out.reshape(x.shape)
